# Initial kernel scaffold; baseline (speedup 1.0000x reference)
import jax
import jax.numpy as jnp
from jax.experimental import pallas as pl


def kernel(x, params):
    raise NotImplementedError("write your pallas kernel here")


# speedup vs baseline: 2.3270x; 2.3270x over previous
import functools

import jax
import jax.numpy as jnp
import numpy as np
from jax.experimental import pallas as pl

NIC = 1
NF = 24
ITERS = 3
WS = 32
TOPK = 8
RANK = 12
DELTA = 10.0
LEAK = 0.2
KS_LIST = (3, 5, 7)
H = 64
W = 64
N = H * W
M2 = WS * WS
NW = (H // WS) * (W // WS)
_BIG = 3.0e38


def _np_local_mask_inf(ks):
    ii = np.arange(M2)
    yi = ii // WS
    xi = ii % WS
    r = (ks - 1) // 2
    m = (np.abs(yi[:, None] - yi[None, :]) <= r) & (np.abs(xi[:, None] - xi[None, :]) <= r)
    return np.where(m, 3.0e38, 0.0).astype(np.float32)


_MASK_INF = {ks: _np_local_mask_inf(ks) for ks in KS_LIST}


def _leaky(x):
    return jnp.where(x >= 0, x, LEAK * x)


def _to_windows(h_flat, c):
    h3 = h_flat.reshape(c, 2, WS, 2, WS)
    return h3.transpose(1, 3, 2, 4, 0).reshape(NW, M2, c)


def _from_windows(hw, c):
    h5 = hw.reshape(2, 2, WS, WS, c)
    return h5.transpose(4, 0, 2, 1, 3).reshape(c, N)


def _im2col(h_flat, c, ks):
    r = (ks - 1) // 2
    img = h_flat.reshape(c, H, W)
    xp = jnp.pad(img, ((0, 0), (r, r), (r, r)), mode='reflect')
    cols = [xp[:, dy:dy + H, dx:dx + W] for dy in range(ks) for dx in range(ks)]
    return jnp.stack(cols, axis=0).reshape(ks * ks * c, N)


def _center_body(x_ref, o_ref):
    x = x_ref[...]
    o_ref[...] = x - jnp.mean(x)


def _center(x_flat):
    return pl.pallas_call(
        _center_body,
        out_shape=jax.ShapeDtypeStruct((1, N), jnp.float32),
    )(x_flat)


def _axpby_body(a_ref, b_ref, x_ref, y_ref, o_ref):
    o_ref[...] = a_ref[0, 0] * x_ref[...] + b_ref[0, 0] * y_ref[...]


def _axpby(a, b, x, y):
    a2 = jnp.asarray(a, jnp.float32).reshape(1, 1)
    b2 = jnp.asarray(b, jnp.float32).reshape(1, 1)
    return pl.pallas_call(
        _axpby_body,
        out_shape=jax.ShapeDtypeStruct(x.shape, jnp.float32),
    )(a2, b2, x, y)


def _conv_body(refs, *, has_bias, has_nl, has_bn, act):
    i = 0
    cols = refs[i]; i += 1
    wr = refs[i]; i += 1
    br = gr = bnr = nlr = None
    if has_bias:
        br = refs[i]; i += 1
    if has_nl:
        nlr = refs[i]; i += 1
    if has_bn:
        gr = refs[i]; i += 1
        bnr = refs[i]; i += 1
    o_ref = refs[i]
    y = jnp.dot(wr[...], cols[...], preferred_element_type=jnp.float32)
    if has_nl:
        y = (nlr[...] + y) * 0.5
    if has_bias:
        y = y + br[...]
    if has_bn:
        mu = jnp.mean(y, axis=1, keepdims=True)
        var = jnp.mean((y - mu) ** 2, axis=1, keepdims=True)
        y = (y - mu) * jax.lax.rsqrt(var + 1e-5) * gr[...] + bnr[...]
    if act:
        y = jnp.where(y >= 0, y, LEAK * y)
    o_ref[...] = y


def _conv(cols, w, bias=None, hnl=None, bn=None, act=False):
    o = w.shape[0]
    ops = [cols, w]
    if bias is not None:
        ops.append(bias.reshape(o, 1))
    if hnl is not None:
        ops.append(hnl)
    if bn is not None:
        ops.append(bn[0].reshape(o, 1))
        ops.append(bn[1].reshape(o, 1))
    body = functools.partial(
        _conv_body, has_bias=bias is not None, has_nl=hnl is not None,
        has_bn=bn is not None, act=act)

    def kern(*refs):
        body(refs)

    return pl.pallas_call(
        kern,
        out_shape=jax.ShapeDtypeStruct((o, N), jnp.float32),
    )(*ops)


def _topk_body(hw_ref, minf_ref, o_ref, *, c):
    hw = hw_ref[0]
    gram = jax.lax.dot_general(hw, hw, (((1,), (1,)), ((), ())),
                               preferred_element_type=jnp.float32)
    sq = jnp.sum(hw * hw, axis=1, keepdims=True)
    g = sq + jnp.transpose(sq) - 2.0 * gram
    g = g + minf_ref[...]
    iota = jax.lax.broadcasted_iota(jnp.int32, (M2, M2), 1)
    for k in range(TOPK):
        mval = jnp.min(g, axis=1, keepdims=True)
        cand = jnp.where(g <= mval, iota, M2 + 1)
        idx = jnp.min(cand, axis=1, keepdims=True)
        o_ref[0, :, k:k + 1] = idx
        g = jnp.where(iota == idx, _BIG, g)


def _topk(hwin, c, ks):
    minf = jnp.asarray(_MASK_INF[ks])
    return pl.pallas_call(
        functools.partial(_topk_body, c=c),
        grid=(NW,),
        in_specs=[
            pl.BlockSpec((1, M2, c), lambda w: (w, 0, 0)),
            pl.BlockSpec((M2, M2), lambda w: (0, 0)),
        ],
        out_specs=pl.BlockSpec((1, M2, TOPK), lambda w: (w, 0, 0)),
        out_shape=jax.ShapeDtypeStruct((NW, M2, TOPK), jnp.int32),
    )(hwin, minf)


def _ecc_body(hw_ref, e_ref, w0_ref, b0_ref, wl_ref, bl_ref, wr_ref, br_ref,
              wk_ref, bk_ref, o_ref, *, cin, cout):
    f32 = jnp.float32
    hw = hw_ref[0]
    e = e_ref[0]
    iota = jax.lax.broadcasted_iota(jnp.int32, (M2, M2), 1)
    msum = (jax.lax.broadcasted_iota(jnp.int32, (RANK * cin, RANK), 0) // cin
            == jax.lax.broadcasted_iota(jnp.int32, (RANK * cin, RANK), 1)
            ).astype(f32)
    mexp = (jax.lax.broadcasted_iota(jnp.int32, (RANK, RANK * cout), 0)
            == jax.lax.broadcasted_iota(jnp.int32, (RANK, RANK * cout), 1) // cout
            ).astype(f32)
    msum2 = (jax.lax.broadcasted_iota(jnp.int32, (RANK * cout, cout), 0) % cout
             == jax.lax.broadcasted_iota(jnp.int32, (RANK * cout, cout), 1)
             ).astype(f32)

    kiota = jax.lax.broadcasted_iota(jnp.int32, (M2, TOPK), 1)

    def body(k, acc):
        idx = jnp.sum(jnp.where(kiota == k, e, 0), axis=1, keepdims=True)
        oh = (iota == idx).astype(f32)
        vertex = jnp.dot(oh, hw, preferred_element_type=f32)
        label = vertex - hw
        theta = label @ w0_ref[...] + b0_ref[...]
        theta = jnp.where(theta >= 0, theta, LEAK * theta)
        gamma = jnp.exp(-jnp.sum(label * label, axis=1, keepdims=True)
                        * (1.0 / DELTA))
        th_l = theta @ wl_ref[...] + bl_ref[...]
        th_r = theta @ wr_ref[...] + br_ref[...]
        kap = theta @ wk_ref[...] + bk_ref[...]
        vt = jnp.concatenate([vertex] * RANK, axis=1)
        s = jnp.dot(th_r * vt, msum, preferred_element_type=f32)
        t = kap * s
        te = jnp.dot(t, mexp, preferred_element_type=f32)
        outk = jnp.dot(th_l * te, msum2, preferred_element_type=f32)
        return acc + gamma * outk

    acc = jax.lax.fori_loop(0, TOPK, body, jnp.zeros((M2, cout), f32))
    o_ref[0] = acc * (1.0 / TOPK)


def _ecc(hwin, edge, p, cin, cout):
    w0t = p['FC0']['w'].T
    b0 = p['FC0']['b'].reshape(1, cin)
    wl = p['FCL']['w'].reshape(cout, RANK, cin).transpose(1, 0, 2)
    wl = wl.reshape(RANK * cout, cin).T
    bl = p['FCL']['b'].reshape(cout, RANK).T.reshape(1, RANK * cout)
    wr = p['FCR']['w'].reshape(cin, RANK, cin).transpose(1, 0, 2)
    wr = wr.reshape(RANK * cin, cin).T
    br = p['FCR']['b'].reshape(cin, RANK).T.reshape(1, RANK * cin)
    wk = p['FCk']['w'].T
    bk = p['FCk']['b'].reshape(1, RANK)
    full = lambda *s: pl.BlockSpec(s, lambda w: (0,) * len(s))
    return pl.pallas_call(
        functools.partial(_ecc_body, cin=cin, cout=cout),
        grid=(NW,),
        in_specs=[
            pl.BlockSpec((1, M2, cin), lambda w: (w, 0, 0)),
            pl.BlockSpec((1, M2, TOPK), lambda w: (w, 0, 0)),
            full(cin, cin), full(1, cin),
            full(cin, RANK * cout), full(1, RANK * cout),
            full(cin, RANK * cin), full(1, RANK * cin),
            full(cin, RANK), full(1, RANK),
        ],
        out_specs=pl.BlockSpec((1, M2, cout), lambda w: (w, 0, 0)),
        out_shape=jax.ShapeDtypeStruct((NW, M2, cout), jnp.float32),
    )(hwin, edge, w0t, b0, wl, bl, wr, br, wk, bk)


def _graph_conv(p, h_flat, edge, c, cout, ks, bn=None, act=True):
    hnl_w = _ecc(_to_windows(h_flat, c), edge, p['ecc'], c, cout)
    hnl = _from_windows(hnl_w, cout)
    cols = _im2col(h_flat, c, ks)
    wmat = p['conv_w'].transpose(0, 2, 3, 1).reshape(cout, ks * ks * c)
    return _conv(cols, wmat, bias=p['bias'].reshape(cout), hnl=hnl,
                 bn=bn, act=act)


def _conv_layer(p, h_flat, c, cout, ks, bn=None, act=False):
    cols = _im2col(h_flat, c, ks)
    wmat = p['w'].transpose(0, 2, 3, 1).reshape(cout, ks * ks * c)
    return _conv(cols, wmat, bias=p['b'], bn=bn, act=act)


def _gclayer(p, h_flat, c, block_type, ks):
    pre = 2 if block_type == 'PRE' else 1
    post = 1 if block_type == 'PRE' else 3
    x = h_flat
    for i in range(pre):
        bn = None
        if block_type != 'PRE':
            bn = (p['bnpre'][i]['g'], p['bnpre'][i]['b'])
        x = _conv_layer(p['conv'][i], x, c, c, ks, bn=bn, act=True)
    edge = _topk(_to_windows(x, c), c, ks)
    for i in range(post):
        bn = None
        if block_type == 'LPF':
            bn = (p['bnpost'][i]['g'], p['bnpost'][i]['b'])
        x = _graph_conv(p['gconv'][i], x, edge, c, c, ks, bn=bn, act=True)
    return x


def kernel(x, params):
    xf = x.reshape(1, N)
    xc = _center(xf)

    nf3 = NF // 3
    feats = []
    for i, ks in enumerate(KS_LIST):
        hi = _conv_layer(params['INCONV'][i], xc, NIC, nf3, ks)
        feats.append(_gclayer(params['PPCONV'][i], hi, nf3, 'PRE', ks))
    z = jnp.concatenate(feats, axis=0)

    hiz = _gclayer(params['HPF'], z, NF, 'HPF', 3)
    alpha = params['alpha']
    beta = params['beta']
    for i in range(ITERS):
        z = _axpby(1.0 - alpha[i], beta[i], z, hiz)
        z = _axpby(1.0, 1.0, z, _gclayer(params['LPF'][i], z, NF, 'LPF', 3))
    z = _axpby(1.0 - alpha[-1], beta[-1], z, hiz)

    edge = _topk(_to_windows(z, NF), NF, 3)
    zo = _graph_conv(params['GCout'], z, edge, NF, NIC, 3, act=False)

    out = _axpby(1.0, 1.0, xf, zo)
    return out.reshape(1, NIC, H, W)
